# baseline (speedup 1.0000x reference)
import jax
import jax.numpy as jnp
from jax import lax
from jax.experimental import pallas as pl
from jax.experimental.pallas import tpu as pltpu
from jax.experimental.pallas import tpu_sc as plsc

VOCAB = 50265
HIDDEN = 768
PAD = 1
EPS = 1e-12
B, S = 4, 2048
TOK = B * S

NUM_WORKERS = 32
PER_W = TOK // NUM_WORKERS
CHUNK = 32
NCHUNK = PER_W // CHUNK
FCHUNK = 32
NFCHUNK = PER_W // FCHUNK
NSLOT = 4
LANES = 16
HVECS = HIDDEN // LANES


def _aux_body(ids_ref, pos_ref, okf_ref, flag_ref):
    mask = (ids_ref[...] != PAD).astype(jnp.int32)
    c = mask
    k = 1
    while k < S:
        shifted = jnp.concatenate(
            [jnp.zeros((B, k), jnp.int32), c[:, : S - k]], axis=1
        )
        c = c + shifted
        k *= 2
    pos = c * mask + PAD
    exp = lax.broadcasted_iota(jnp.int32, (B, S), 1) + 2
    ok = (pos == exp).astype(jnp.float32)
    clean = jnp.min(ok)
    pos_ref[...] = pos
    okf_ref[...] = ok * clean
    flag_ref[...] = jnp.full((1, 1), clean, jnp.float32)


def _pos_aux(input_ids):
    return pl.pallas_call(
        _aux_body,
        out_shape=(
            jax.ShapeDtypeStruct((B, S), jnp.int32),
            jax.ShapeDtypeStruct((B, S), jnp.float32),
            jax.ShapeDtypeStruct((1, 1), jnp.float32),
        ),
    )(input_ids)


def _sc_fast_body(wids_hbm, wtab_hbm, out_hbm, *scr):
    cid = lax.axis_index("c")
    sid = lax.axis_index("s")
    wid = cid * 16 + sid
    base = wid * PER_W

    idxw = scr[0:NSLOT]
    bw = scr[NSLOT:2 * NSLOT]
    sw = scr[2 * NSLOT:3 * NSLOT]
    so = scr[3 * NSLOT:4 * NSLOT]

    pend_g = [None] * NSLOT
    pend_s = [None] * NSLOT
    for j in range(NFCHUNK + 1):
        s = j % NSLOT
        if j < NFCHUNK:
            start = base + j * FCHUNK
            if pend_s[s] is not None:
                pend_s[s].wait()
                pend_s[s] = None
            pltpu.sync_copy(wids_hbm.at[pl.ds(start, FCHUNK)], idxw[s])
            pend_g[s] = pltpu.async_copy(wtab_hbm.at[idxw[s]], bw[s], sw[s])
        if j >= 1:
            t = (j - 1) % NSLOT
            pend_g[t].wait()
            pend_s[t] = pltpu.async_copy(
                bw[t], out_hbm.at[pl.ds(base + (j - 1) * FCHUNK, FCHUNK)],
                so[t]
            )
    for s in range(NSLOT):
        if pend_s[s] is not None:
            pend_s[s].wait()


def _sc_fast(input_ids_flat, word_emb):
    mesh = plsc.VectorSubcoreMesh(core_axis_name="c", subcore_axis_name="s")
    f = pl.kernel(
        _sc_fast_body,
        out_type=jax.ShapeDtypeStruct((TOK, HIDDEN), jnp.float32),
        mesh=mesh,
        scratch_types=(
            [pltpu.VMEM((FCHUNK,), jnp.int32)] * NSLOT
            + [pltpu.VMEM((FCHUNK, HIDDEN), jnp.float32)] * NSLOT
            + [pltpu.SemaphoreType.DMA] * (2 * NSLOT)
        ),
    )
    return f(input_ids_flat, word_emb)


def _sc_full_body(wids_hbm, pids_hbm, wtab_hbm, ptab_hbm, out_hbm,
                  idxw0, idxp0, bw0, bp0, sw0, sp0):
    cid = lax.axis_index("c")
    sid = lax.axis_index("s")
    wid = cid * 16 + sid
    base = wid * PER_W

    def chunk_body(j, _):
        start = base + j * CHUNK
        pltpu.sync_copy(wids_hbm.at[pl.ds(start, CHUNK)], idxw0)
        pltpu.sync_copy(pids_hbm.at[pl.ds(start, CHUNK)], idxp0)
        cw = pltpu.async_copy(wtab_hbm.at[idxw0], bw0, sw0)
        cp = pltpu.async_copy(ptab_hbm.at[idxp0], bp0, sp0)
        cw.wait()
        cp.wait()

        def row_body(i, _):
            for k in range(HVECS):
                x = bp0[i, pl.ds(k * LANES, LANES)]
                plsc.addupdate(bw0.at[i, pl.ds(k * LANES, LANES)], x)
            return 0

        lax.fori_loop(0, CHUNK, row_body, 0)
        pltpu.sync_copy(bw0, out_hbm.at[pl.ds(start, CHUNK)])
        return 0

    lax.fori_loop(0, NCHUNK, chunk_body, 0)


def _sc_full(input_ids_flat, pos_ids_flat, word_emb, pos_emb):
    mesh = plsc.VectorSubcoreMesh(core_axis_name="c", subcore_axis_name="s")
    f = pl.kernel(
        _sc_full_body,
        out_type=jax.ShapeDtypeStruct((TOK, HIDDEN), jnp.float32),
        mesh=mesh,
        scratch_types=[
            pltpu.VMEM((CHUNK,), jnp.int32),
            pltpu.VMEM((CHUNK,), jnp.int32),
            pltpu.VMEM((CHUNK, HIDDEN), jnp.float32),
            pltpu.VMEM((CHUNK, HIDDEN), jnp.float32),
            pltpu.SemaphoreType.DMA,
            pltpu.SemaphoreType.DMA,
        ],
    )
    return f(input_ids_flat, pos_ids_flat, word_emb, pos_emb)


SB = 512
NSB = S // SB
ROWS = B * SB


def _ln_body(sum_ref, tt_ref, ip_ref, ok_ref, type_ref, item_ref,
             g_ref, b_ref, posa_ref, posb_ref, out_ref):
    tt = tt_ref[0, 0]
    ip = ip_ref[0, 0]
    oh_t = (tt[:, None] == lax.broadcasted_iota(jnp.int32, (ROWS, 4), 1)
            ).astype(jnp.bfloat16)
    oh_i = (ip[:, None] == lax.broadcasted_iota(jnp.int32, (ROWS, 32), 1)
            ).astype(jnp.bfloat16)
    small = jnp.dot(oh_t, type_ref[...].astype(jnp.bfloat16),
                    preferred_element_type=jnp.float32)
    small = small + jnp.dot(oh_i, item_ref[...].astype(jnp.bfloat16),
                            preferred_element_type=jnp.float32)
    posd = jnp.concatenate(
        [posa_ref[pl.ds(2, SB - 2), :], posb_ref[pl.ds(0, 2), :]], axis=0
    )
    x = sum_ref[...] + ok_ref[0][:, :, None] * posd[None, :, :]
    x = x.reshape(ROWS, HIDDEN) + small
    mean = jnp.mean(x, axis=1, keepdims=True)
    d = x - mean
    var = jnp.mean(d * d, axis=1, keepdims=True)
    y = d * lax.rsqrt(var + EPS)
    y = y * g_ref[...] + b_ref[...]
    out_ref[...] = y.reshape(B, SB, HIDDEN)


def _ln(sum3, tt_r, ip_r, ok_r, type_emb, item_emb, gamma2, beta2, pos_emb):
    return pl.pallas_call(
        _ln_body,
        grid=(NSB,),
        in_specs=[
            pl.BlockSpec((B, SB, HIDDEN), lambda i: (0, i, 0)),
            pl.BlockSpec((1, 1, ROWS), lambda i: (i, 0, 0)),
            pl.BlockSpec((1, 1, ROWS), lambda i: (i, 0, 0)),
            pl.BlockSpec((1, B, SB), lambda i: (i, 0, 0)),
            pl.BlockSpec((4, HIDDEN), lambda i: (0, 0)),
            pl.BlockSpec((32, HIDDEN), lambda i: (0, 0)),
            pl.BlockSpec((1, HIDDEN), lambda i: (0, 0)),
            pl.BlockSpec((1, HIDDEN), lambda i: (0, 0)),
            pl.BlockSpec((SB, HIDDEN), lambda i: (i, 0)),
            pl.BlockSpec((8, HIDDEN), lambda i: ((SB // 8) * (i + 1), 0)),
        ],
        out_specs=pl.BlockSpec((B, SB, HIDDEN), lambda i: (0, i, 0)),
        out_shape=jax.ShapeDtypeStruct((B, S, HIDDEN), jnp.float32),
    )(sum3, tt_r, ip_r, ok_r, type_emb, item_emb, gamma2, beta2,
      pos_emb, pos_emb)


def kernel(input_ids, token_type_ids, item_position_ids, word_emb, pos_emb,
           type_emb, item_emb, ln_gamma, ln_beta):
    pos_ids, okf_ln, flag = _pos_aux(input_ids)
    shifted = flag[0, 0] < 0.5
    wids = input_ids.reshape(TOK)

    sum_w = lax.cond(
        shifted,
        lambda: _sc_full(wids, pos_ids.reshape(TOK), word_emb, pos_emb),
        lambda: _sc_fast(wids, word_emb),
    )

    tt_r = (token_type_ids.reshape(B, NSB, SB).transpose(1, 0, 2)
            .reshape(NSB, 1, ROWS))
    ip_r = (item_position_ids.reshape(B, NSB, SB).transpose(1, 0, 2)
            .reshape(NSB, 1, ROWS))
    ok_r = okf_ln.reshape(B, NSB, SB).transpose(1, 0, 2)
    return _ln(
        sum_w.reshape(B, S, HIDDEN), tt_r, ip_r, ok_r, type_emb, item_emb,
        ln_gamma.reshape(1, HIDDEN), ln_beta.reshape(1, HIDDEN), pos_emb,
    )


# speedup vs baseline: 1.0083x; 1.0083x over previous
import jax
import jax.numpy as jnp
from jax import lax
from jax.experimental import pallas as pl
from jax.experimental.pallas import tpu as pltpu
from jax.experimental.pallas import tpu_sc as plsc

VOCAB = 50265
HIDDEN = 768
PAD = 1
EPS = 1e-12
B, S = 4, 2048
TOK = B * S

NUM_WORKERS = 32
PER_W = TOK // NUM_WORKERS
CHUNK = 32
NCHUNK = PER_W // CHUNK
FCHUNK = 64
NFCHUNK = PER_W // FCHUNK
NSLOT = 2
LANES = 16
HVECS = HIDDEN // LANES


def _aux_body(ids_ref, pos_ref, okf_ref, flag_ref):
    mask = (ids_ref[...] != PAD).astype(jnp.int32)
    c = mask
    k = 1
    while k < S:
        shifted = jnp.concatenate(
            [jnp.zeros((B, k), jnp.int32), c[:, : S - k]], axis=1
        )
        c = c + shifted
        k *= 2
    pos = c * mask + PAD
    exp = lax.broadcasted_iota(jnp.int32, (B, S), 1) + 2
    ok = (pos == exp).astype(jnp.float32)
    clean = jnp.min(ok)
    pos_ref[...] = pos
    okf_ref[...] = ok * clean
    flag_ref[...] = jnp.full((1, 1), clean, jnp.float32)


def _pos_aux(input_ids):
    return pl.pallas_call(
        _aux_body,
        out_shape=(
            jax.ShapeDtypeStruct((B, S), jnp.int32),
            jax.ShapeDtypeStruct((B, S), jnp.float32),
            jax.ShapeDtypeStruct((1, 1), jnp.float32),
        ),
    )(input_ids)


def _sc_fast_body(wids_hbm, wtab_hbm, out_hbm, *scr):
    cid = lax.axis_index("c")
    sid = lax.axis_index("s")
    wid = cid * 16 + sid
    base = wid * PER_W

    idxw = scr[0:NSLOT]
    bw = scr[NSLOT:2 * NSLOT]
    sw = scr[2 * NSLOT:3 * NSLOT]
    so = scr[3 * NSLOT:4 * NSLOT]

    pend_g = [None] * NSLOT
    pend_s = [None] * NSLOT
    for j in range(NFCHUNK + 1):
        s = j % NSLOT
        if j < NFCHUNK:
            start = base + j * FCHUNK
            if pend_s[s] is not None:
                pend_s[s].wait()
                pend_s[s] = None
            pltpu.sync_copy(wids_hbm.at[pl.ds(start, FCHUNK)], idxw[s])
            pend_g[s] = pltpu.async_copy(wtab_hbm.at[idxw[s]], bw[s], sw[s])
        if j >= 1:
            t = (j - 1) % NSLOT
            pend_g[t].wait()
            pend_s[t] = pltpu.async_copy(
                bw[t], out_hbm.at[pl.ds(base + (j - 1) * FCHUNK, FCHUNK)],
                so[t]
            )
    for s in range(NSLOT):
        if pend_s[s] is not None:
            pend_s[s].wait()


def _sc_fast(input_ids_flat, word_emb):
    mesh = plsc.VectorSubcoreMesh(core_axis_name="c", subcore_axis_name="s")
    f = pl.kernel(
        _sc_fast_body,
        out_type=jax.ShapeDtypeStruct((TOK, HIDDEN), jnp.float32),
        mesh=mesh,
        scratch_types=(
            [pltpu.VMEM((FCHUNK,), jnp.int32)] * NSLOT
            + [pltpu.VMEM((FCHUNK, HIDDEN), jnp.float32)] * NSLOT
            + [pltpu.SemaphoreType.DMA] * (2 * NSLOT)
        ),
    )
    return f(input_ids_flat, word_emb)


def _sc_full_body(wids_hbm, pids_hbm, wtab_hbm, ptab_hbm, out_hbm,
                  idxw0, idxp0, bw0, bp0, sw0, sp0):
    cid = lax.axis_index("c")
    sid = lax.axis_index("s")
    wid = cid * 16 + sid
    base = wid * PER_W

    def chunk_body(j, _):
        start = base + j * CHUNK
        pltpu.sync_copy(wids_hbm.at[pl.ds(start, CHUNK)], idxw0)
        pltpu.sync_copy(pids_hbm.at[pl.ds(start, CHUNK)], idxp0)
        cw = pltpu.async_copy(wtab_hbm.at[idxw0], bw0, sw0)
        cp = pltpu.async_copy(ptab_hbm.at[idxp0], bp0, sp0)
        cw.wait()
        cp.wait()

        def row_body(i, _):
            for k in range(HVECS):
                x = bp0[i, pl.ds(k * LANES, LANES)]
                plsc.addupdate(bw0.at[i, pl.ds(k * LANES, LANES)], x)
            return 0

        lax.fori_loop(0, CHUNK, row_body, 0)
        pltpu.sync_copy(bw0, out_hbm.at[pl.ds(start, CHUNK)])
        return 0

    lax.fori_loop(0, NCHUNK, chunk_body, 0)


def _sc_full(input_ids_flat, pos_ids_flat, word_emb, pos_emb):
    mesh = plsc.VectorSubcoreMesh(core_axis_name="c", subcore_axis_name="s")
    f = pl.kernel(
        _sc_full_body,
        out_type=jax.ShapeDtypeStruct((TOK, HIDDEN), jnp.float32),
        mesh=mesh,
        scratch_types=[
            pltpu.VMEM((CHUNK,), jnp.int32),
            pltpu.VMEM((CHUNK,), jnp.int32),
            pltpu.VMEM((CHUNK, HIDDEN), jnp.float32),
            pltpu.VMEM((CHUNK, HIDDEN), jnp.float32),
            pltpu.SemaphoreType.DMA,
            pltpu.SemaphoreType.DMA,
        ],
    )
    return f(input_ids_flat, pos_ids_flat, word_emb, pos_emb)


SB = 512
NSB = S // SB
ROWS = B * SB


def _ln_body(sum_ref, tt_ref, ip_ref, ok_ref, type_ref, item_ref,
             g_ref, b_ref, posa_ref, posb_ref, out_ref):
    tt = tt_ref[0, 0]
    ip = ip_ref[0, 0]
    oh_t = (tt[:, None] == lax.broadcasted_iota(jnp.int32, (ROWS, 4), 1)
            ).astype(jnp.bfloat16)
    oh_i = (ip[:, None] == lax.broadcasted_iota(jnp.int32, (ROWS, 32), 1)
            ).astype(jnp.bfloat16)
    small = jnp.dot(oh_t, type_ref[...].astype(jnp.bfloat16),
                    preferred_element_type=jnp.float32)
    small = small + jnp.dot(oh_i, item_ref[...].astype(jnp.bfloat16),
                            preferred_element_type=jnp.float32)
    posd = jnp.concatenate(
        [posa_ref[pl.ds(2, SB - 2), :], posb_ref[pl.ds(0, 2), :]], axis=0
    )
    x = sum_ref[...] + ok_ref[0][:, :, None] * posd[None, :, :]
    x = x.reshape(ROWS, HIDDEN) + small
    mean = jnp.mean(x, axis=1, keepdims=True)
    d = x - mean
    var = jnp.mean(d * d, axis=1, keepdims=True)
    y = d * lax.rsqrt(var + EPS)
    y = y * g_ref[...] + b_ref[...]
    out_ref[...] = y.reshape(B, SB, HIDDEN)


def _ln(sum3, tt_r, ip_r, ok_r, type_emb, item_emb, gamma2, beta2, pos_emb):
    return pl.pallas_call(
        _ln_body,
        grid=(NSB,),
        in_specs=[
            pl.BlockSpec((B, SB, HIDDEN), lambda i: (0, i, 0)),
            pl.BlockSpec((1, 1, ROWS), lambda i: (i, 0, 0)),
            pl.BlockSpec((1, 1, ROWS), lambda i: (i, 0, 0)),
            pl.BlockSpec((1, B, SB), lambda i: (i, 0, 0)),
            pl.BlockSpec((4, HIDDEN), lambda i: (0, 0)),
            pl.BlockSpec((32, HIDDEN), lambda i: (0, 0)),
            pl.BlockSpec((1, HIDDEN), lambda i: (0, 0)),
            pl.BlockSpec((1, HIDDEN), lambda i: (0, 0)),
            pl.BlockSpec((SB, HIDDEN), lambda i: (i, 0)),
            pl.BlockSpec((8, HIDDEN), lambda i: ((SB // 8) * (i + 1), 0)),
        ],
        out_specs=pl.BlockSpec((B, SB, HIDDEN), lambda i: (0, i, 0)),
        out_shape=jax.ShapeDtypeStruct((B, S, HIDDEN), jnp.float32),
    )(sum3, tt_r, ip_r, ok_r, type_emb, item_emb, gamma2, beta2,
      pos_emb, pos_emb)


def kernel(input_ids, token_type_ids, item_position_ids, word_emb, pos_emb,
           type_emb, item_emb, ln_gamma, ln_beta):
    pos_ids, okf_ln, flag = _pos_aux(input_ids)
    shifted = flag[0, 0] < 0.5
    wids = input_ids.reshape(TOK)

    sum_w = lax.cond(
        shifted,
        lambda: _sc_full(wids, pos_ids.reshape(TOK), word_emb, pos_emb),
        lambda: _sc_fast(wids, word_emb),
    )

    tt_r = (token_type_ids.reshape(B, NSB, SB).transpose(1, 0, 2)
            .reshape(NSB, 1, ROWS))
    ip_r = (item_position_ids.reshape(B, NSB, SB).transpose(1, 0, 2)
            .reshape(NSB, 1, ROWS))
    ok_r = okf_ln.reshape(B, NSB, SB).transpose(1, 0, 2)
    return _ln(
        sum_w.reshape(B, S, HIDDEN), tt_r, ip_r, ok_r, type_emb, item_emb,
        ln_gamma.reshape(1, HIDDEN), ln_beta.reshape(1, HIDDEN), pos_emb,
    )
